# Initial kernel scaffold; baseline (speedup 1.0000x reference)
import jax
import jax.numpy as jnp
from jax.experimental import pallas as pl


def kernel(x, sa1_W0, sa1_b0, sa1_W1, sa1_b1, sa1_W2, sa1_b2, sa2_W0, sa2_b0, sa2_W1, sa2_b1, sa2_W2, sa2_b2, sa3_W0, sa3_b0, sa3_W1, sa3_b1, sa3_W2, sa3_b2):
    raise NotImplementedError("write your pallas kernel here")


# speedup vs baseline: 10.2638x; 10.2638x over previous
import functools

import jax
import jax.numpy as jnp
from jax import lax
from jax.experimental import pallas as pl
from jax.experimental.pallas import tpu as pltpu
from jax.experimental.pallas import tpu_sc as plsc

F32 = jnp.float32
_SC_CORES = 2
_SC_SUBCORES = 16
_SC_WORKERS = _SC_CORES * _SC_SUBCORES


def _fps_body(xt_ref, out_ref, lane_ref, col_ref):
  B, _, N = xt_ref.shape
  M = out_ref.shape[1]
  xs = xt_ref[:, 0, :]
  ys = xt_ref[:, 1, :]
  zs = xt_ref[:, 2, :]
  lane_ref[...] = lax.broadcasted_iota(jnp.int32, (B, N), 1)
  col_ref[...] = lax.broadcasted_iota(jnp.int32, (B, M), 1)
  lane = lane_ref[...]
  col = col_ref[...]

  def step(s, carry):
    far, dists, out = carry
    sel = (col == s).astype(jnp.int32)
    out = out + sel * (far - out)
    oh = (lane == far).astype(F32)
    cx = jnp.sum(xs * oh, axis=1, keepdims=True)
    cy = jnp.sum(ys * oh, axis=1, keepdims=True)
    cz = jnp.sum(zs * oh, axis=1, keepdims=True)
    d = (xs - cx) ** 2 + (ys - cy) ** 2 + (zs - cz) ** 2
    dists = jnp.minimum(dists, d)
    mx = jnp.max(dists, axis=1, keepdims=True)
    far = jnp.min(lane + (dists != mx).astype(jnp.int32) * N,
                  axis=1, keepdims=True)
    return far, dists, out

  far0 = col[:, :1] * 0
  d0 = jnp.maximum(xs * 0.0, 1e10)
  out0 = col * 0
  _, _, out = lax.fori_loop(0, M, step, (far0, d0, out0))
  col_ref[...] = lax.broadcasted_iota(jnp.int32, (B, M), 0)
  out_ref[...] = out + col_ref[...] * N


def _fps(xt, m):
  B, _, N = xt.shape
  return pl.pallas_call(
      _fps_body,
      out_shape=jax.ShapeDtypeStruct((B, m), jnp.int32),
      scratch_shapes=[
          pltpu.VMEM((B, N), jnp.int32),
          pltpu.VMEM((B, m), jnp.int32),
      ],
  )(xt)


def _bq_body(k, r2, xt_ref, c_ref, out_ref, lane_ref):
  N = xt_ref.shape[2]
  Mb = c_ref.shape[1]
  b = pl.program_id(0)
  xs = xt_ref[0, 0, :][None, :]
  ys = xt_ref[0, 1, :][None, :]
  zs = xt_ref[0, 2, :][None, :]
  cx = c_ref[0, :, 0:1]
  cy = c_ref[0, :, 1:2]
  cz = c_ref[0, :, 2:3]
  d = (cx - xs) ** 2 + (cy - ys) ** 2 + (cz - zs) ** 2
  lane_ref[...] = lax.broadcasted_iota(jnp.int32, (Mb, N), 1)
  lane = lane_ref[...]
  midx = lane + (d > r2).astype(jnp.int32) * N
  cols = []
  for _ in range(k):
    cur = jnp.min(midx, axis=1, keepdims=True)
    midx = jnp.where(midx == cur, 2 * N, midx)
    cols.append(cur)
  idx = jnp.concatenate(cols, axis=1)
  pad = (idx >= N).astype(jnp.int32)
  idx = idx + pad * (idx[:, 0:1] - idx)
  out_ref[0] = idx + b * N


def _ball_query(xt, c, k, r2, mb):
  B, _, N = xt.shape
  M = c.shape[1]
  Dc = c.shape[2]
  body = functools.partial(_bq_body, k, r2)
  return pl.pallas_call(
      body,
      grid=(B, M // mb),
      in_specs=[
          pl.BlockSpec((1, 3, N), lambda b, m: (b, 0, 0)),
          pl.BlockSpec((1, mb, Dc), lambda b, m: (b, m, 0)),
      ],
      out_specs=pl.BlockSpec((1, mb, k), lambda b, m: (b, m, 0)),
      out_shape=jax.ShapeDtypeStruct((B, M, k), jnp.int32),
      scratch_shapes=[pltpu.VMEM((mb, N), jnp.int32)],
      compiler_params=pltpu.CompilerParams(
          dimension_semantics=("parallel", "parallel")),
  )(xt, c)


def _sc_gather(table, idx):
  R = idx.shape[0]
  D = table.shape[1]
  bpw = R // _SC_WORKERS
  ch = min(bpw, 128)
  nch = bpw // ch
  mesh = plsc.VectorSubcoreMesh(core_axis_name="c", subcore_axis_name="s")

  @functools.partial(
      pl.kernel,
      mesh=mesh,
      out_type=jax.ShapeDtypeStruct((R, D), F32),
      scratch_types=[
          pltpu.VMEM((ch,), jnp.int32),
          pltpu.VMEM((ch, D), F32),
          pltpu.SemaphoreType.DMA,
      ],
  )
  def k(table_hbm, idx_hbm, out_hbm, idx_v, rows_v, sem):
    wid = lax.axis_index("s") * _SC_CORES + lax.axis_index("c")
    base = wid * bpw

    def body(i, carry):
      off = pl.multiple_of(base + i * ch, 8)
      pltpu.sync_copy(idx_hbm.at[pl.ds(off, ch)], idx_v)
      pltpu.async_copy(table_hbm.at[idx_v], rows_v, sem).wait()
      pltpu.sync_copy(rows_v, out_hbm.at[pl.ds(off, ch)])
      return carry

    lax.fori_loop(0, nch, body, 0)

  return k(table, idx)


def _dot_t(x, w):
  return lax.dot_general(x, w, (((1,), (1,)), ((), ())),
                         precision=lax.Precision.HIGHEST)


def _relu(x):
  return jnp.maximum(x, 0.0)


def _mlp_pool_body(g_ref, c_ref, w0_ref, b0_ref, w1_ref, b1_ref, w2_ref,
                   b2_ref, out_ref, *, in_dim, out_pad):
  Mb, K, Dg = g_ref.shape[1], g_ref.shape[2], g_ref.shape[3]
  g = g_ref[0]
  cc = c_ref[0][:, :3]
  coords = g[:, :, :3] - cc[:, None, :]
  if in_dim > 3:
    feats = g[:, :, 3:in_dim]
    x = jnp.concatenate([coords, feats], axis=-1)
  else:
    x = coords
  x = x.reshape(Mb * K, in_dim)
  h = _relu(_dot_t(x, w0_ref[...]) + b0_ref[...])
  h = _relu(_dot_t(h, w1_ref[...]) + b1_ref[...])
  h = _relu(_dot_t(h, w2_ref[...]) + b2_ref[...])
  f = jnp.max(h.reshape(Mb, K, h.shape[-1]), axis=1)
  if out_pad:
    out = jnp.concatenate(
        [cc, f, jnp.zeros((Mb, out_pad), F32)], axis=-1)
  else:
    out = f
  out_ref[0] = out


def _mlp_pool(g, c, params, in_dim, out_pad, mb):
  B, M, K, Dg = g.shape
  Dc = c.shape[2]
  w0, b0, w1, b1, w2, b2 = params
  o = w2.shape[0]
  dout = (3 + o + out_pad) if out_pad else o
  body = functools.partial(_mlp_pool_body, in_dim=in_dim, out_pad=out_pad)
  wspec = lambda a: pl.BlockSpec(a.shape, lambda b, m: (0,) * a.ndim)
  return pl.pallas_call(
      body,
      grid=(B, M // mb),
      in_specs=[
          pl.BlockSpec((1, mb, K, Dg), lambda b, m: (b, m, 0, 0)),
          pl.BlockSpec((1, mb, Dc), lambda b, m: (b, m, 0)),
          wspec(w0), wspec(b0), wspec(w1), wspec(b1), wspec(w2), wspec(b2),
      ],
      out_specs=pl.BlockSpec((1, mb, dout), lambda b, m: (b, m, 0)),
      out_shape=jax.ShapeDtypeStruct((B, M, dout), F32),
      compiler_params=pltpu.CompilerParams(
          dimension_semantics=("parallel", "parallel")),
  )(g, c, w0, b0, w1, b1, w2, b2)


def _global_mlp_body(c_ref, f_ref, w0_ref, b0_ref, w1_ref, b1_ref, w2_ref,
                     b2_ref, out_ref):
  x = jnp.concatenate([c_ref[0][:, :3], f_ref[0]], axis=-1)
  h = _relu(_dot_t(x, w0_ref[...]) + b0_ref[...])
  h = _relu(_dot_t(h, w1_ref[...]) + b1_ref[...])
  h = _relu(_dot_t(h, w2_ref[...]) + b2_ref[...])
  out_ref[0, 0] = jnp.max(h, axis=0)


def _global_mlp(c, f, params):
  B, M, Dc = c.shape
  Fd = f.shape[2]
  w0, b0, w1, b1, w2, b2 = params
  o = w2.shape[0]
  wspec = lambda a: pl.BlockSpec(a.shape, lambda b: (0,) * a.ndim)
  return pl.pallas_call(
      _global_mlp_body,
      grid=(B,),
      in_specs=[
          pl.BlockSpec((1, M, Dc), lambda b: (b, 0, 0)),
          pl.BlockSpec((1, M, Fd), lambda b: (b, 0, 0)),
          wspec(w0), wspec(b0), wspec(w1), wspec(b1), wspec(w2), wspec(b2),
      ],
      out_specs=pl.BlockSpec((1, 1, o), lambda b: (b, 0, 0)),
      out_shape=jax.ShapeDtypeStruct((B, 1, o), F32),
      compiler_params=pltpu.CompilerParams(
          dimension_semantics=("parallel",)),
  )(c, f, w0, b0, w1, b1, w2, b2).reshape(B, o)


def kernel(x, sa1_W0, sa1_b0, sa1_W1, sa1_b1, sa1_W2, sa1_b2,
           sa2_W0, sa2_b0, sa2_W1, sa2_b1, sa2_W2, sa2_b2,
           sa3_W0, sa3_b0, sa3_W1, sa3_b1, sa3_W2, sa3_b2):
  B, N, _ = x.shape
  M1, K1, M2, K2 = 512, 32, 128, 64
  r1sq, r2sq = 0.2 * 0.2, 0.4 * 0.4

  p1 = (sa1_W0, sa1_b0.reshape(1, -1), sa1_W1, sa1_b1.reshape(1, -1),
        sa1_W2, sa1_b2.reshape(1, -1))
  p2 = (sa2_W0, sa2_b0.reshape(1, -1), sa2_W1, sa2_b1.reshape(1, -1),
        sa2_W2, sa2_b2.reshape(1, -1))
  p3 = (sa3_W0, sa3_b0.reshape(1, -1), sa3_W1, sa3_b1.reshape(1, -1),
        sa3_W2, sa3_b2.reshape(1, -1))

  xt = jnp.transpose(x, (0, 2, 1))
  x_pad = jnp.pad(x, ((0, 0), (0, 0), (0, 125))).reshape(B * N, 128)

  i1 = _fps(xt, M1)
  c1 = _sc_gather(x_pad, i1.reshape(-1))
  c1r = c1.reshape(B, M1, 128)
  idx1 = _ball_query(xt, c1r, K1, r1sq, 128)
  g1 = _sc_gather(x_pad, idx1.reshape(-1))
  g1 = g1.reshape(B, M1, K1, 128)
  cf1 = _mlp_pool(g1, c1r, p1, in_dim=3, out_pad=125, mb=128)
  cf1_flat = cf1.reshape(B * M1, 256)

  c1t = jnp.transpose(c1r[:, :, :3], (0, 2, 1))
  i2 = _fps(c1t, M2)
  c2 = _sc_gather(cf1_flat, i2.reshape(-1))
  c2r = c2.reshape(B, M2, 256)
  idx2 = _ball_query(c1t, c2r, K2, r2sq, 128)
  g2 = _sc_gather(cf1_flat, idx2.reshape(-1))
  g2 = g2.reshape(B, M2, K2, 256)
  f2 = _mlp_pool(g2, c2r, p2, in_dim=131, out_pad=0, mb=128)

  return _global_mlp(c2r, f2, p3)


# speedup vs baseline: 11.7975x; 1.1494x over previous
import functools

import jax
import jax.numpy as jnp
from jax import lax
from jax.experimental import pallas as pl
from jax.experimental.pallas import tpu as pltpu
from jax.experimental.pallas import tpu_sc as plsc

F32 = jnp.float32
_SC_CORES = 2
_SC_SUBCORES = 16
_SC_WORKERS = _SC_CORES * _SC_SUBCORES


def _fps_body(xt_ref, out_ref, lane_ref, col_ref):
  B, _, N = xt_ref.shape
  M = out_ref.shape[1]
  xs = xt_ref[:, 0, :]
  ys = xt_ref[:, 1, :]
  zs = xt_ref[:, 2, :]
  lane_ref[...] = lax.broadcasted_iota(jnp.int32, (B, N), 1)
  col_ref[...] = lax.broadcasted_iota(jnp.int32, (B, M), 1)
  lane = lane_ref[...]
  col = col_ref[...]

  def step(s, carry):
    far, dists, out = carry
    sel = (col == s).astype(jnp.int32)
    out = out + sel * (far - out)
    oh = (lane == far).astype(F32)
    cx = jnp.sum(xs * oh, axis=1, keepdims=True)
    cy = jnp.sum(ys * oh, axis=1, keepdims=True)
    cz = jnp.sum(zs * oh, axis=1, keepdims=True)
    d = (xs - cx) ** 2 + (ys - cy) ** 2 + (zs - cz) ** 2
    dists = jnp.minimum(dists, d)
    mx = jnp.max(dists, axis=1, keepdims=True)
    far = jnp.min(lane + (dists != mx).astype(jnp.int32) * N,
                  axis=1, keepdims=True)
    return far, dists, out

  far0 = col[:, :1] * 0
  d0 = jnp.maximum(xs * 0.0, 1e10)
  out0 = col * 0
  _, _, out = lax.fori_loop(0, M, step, (far0, d0, out0))
  col_ref[...] = lax.broadcasted_iota(jnp.int32, (B, M), 0)
  out_ref[...] = out + col_ref[...] * N


def _fps(xt, m):
  B, _, N = xt.shape
  return pl.pallas_call(
      _fps_body,
      out_shape=jax.ShapeDtypeStruct((B, m), jnp.int32),
      scratch_shapes=[
          pltpu.VMEM((B, N), jnp.int32),
          pltpu.VMEM((B, m), jnp.int32),
      ],
  )(xt)


def _bq_body(k, r2, xt_ref, c_ref, out_ref, lane_ref):
  N = xt_ref.shape[2]
  Mb = c_ref.shape[1]
  b = pl.program_id(0)
  xs = xt_ref[0, 0, :][None, :]
  ys = xt_ref[0, 1, :][None, :]
  zs = xt_ref[0, 2, :][None, :]
  cx = c_ref[0, :, 0:1]
  cy = c_ref[0, :, 1:2]
  cz = c_ref[0, :, 2:3]
  d = (cx - xs) ** 2 + (cy - ys) ** 2 + (cz - zs) ** 2
  lane_ref[...] = lax.broadcasted_iota(jnp.int32, (Mb, N), 1)
  lane = lane_ref[...].astype(F32)
  midx = lane + (d > r2).astype(F32) * N
  big = 3.0 * N
  cur = jnp.min(midx, axis=1, keepdims=True)
  cols = [cur]
  for _ in range(k - 1):
    cur = jnp.min(jnp.where(midx > cur, midx, big), axis=1, keepdims=True)
    cols.append(cur)
  idx = jnp.concatenate(cols, axis=1)
  pad = (idx >= N).astype(F32)
  idx = idx + pad * (idx[:, 0:1] - idx)
  out_ref[0] = idx.astype(jnp.int32) + b * N


def _ball_query(xt, c, k, r2, mb):
  B, _, N = xt.shape
  M = c.shape[1]
  Dc = c.shape[2]
  body = functools.partial(_bq_body, k, r2)
  return pl.pallas_call(
      body,
      grid=(B, M // mb),
      in_specs=[
          pl.BlockSpec((1, 3, N), lambda b, m: (b, 0, 0)),
          pl.BlockSpec((1, mb, Dc), lambda b, m: (b, m, 0)),
      ],
      out_specs=pl.BlockSpec((1, mb, k), lambda b, m: (b, m, 0)),
      out_shape=jax.ShapeDtypeStruct((B, M, k), jnp.int32),
      scratch_shapes=[pltpu.VMEM((mb, N), jnp.int32)],
      compiler_params=pltpu.CompilerParams(
          dimension_semantics=("parallel", "parallel")),
  )(xt, c)


def _sc_gather(table, idx):
  R = idx.shape[0]
  D = table.shape[1]
  bpw = R // _SC_WORKERS
  ch = min(bpw, 128)
  nch = bpw // ch
  mesh = plsc.VectorSubcoreMesh(core_axis_name="c", subcore_axis_name="s")

  @functools.partial(
      pl.kernel,
      mesh=mesh,
      out_type=jax.ShapeDtypeStruct((R, D), F32),
      scratch_types=[
          pltpu.VMEM((ch,), jnp.int32),
          pltpu.VMEM((ch, D), F32),
          pltpu.SemaphoreType.DMA,
      ],
  )
  def k(table_hbm, idx_hbm, out_hbm, idx_v, rows_v, sem):
    wid = lax.axis_index("s") * _SC_CORES + lax.axis_index("c")
    base = wid * bpw

    def body(i, carry):
      off = pl.multiple_of(base + i * ch, 8)
      pltpu.sync_copy(idx_hbm.at[pl.ds(off, ch)], idx_v)
      pltpu.async_copy(table_hbm.at[idx_v], rows_v, sem).wait()
      pltpu.sync_copy(rows_v, out_hbm.at[pl.ds(off, ch)])
      return carry

    lax.fori_loop(0, nch, body, 0)

  return k(table, idx)


def _dot_t(x, w):
  return lax.dot_general(x, w, (((1,), (1,)), ((), ())),
                         precision=lax.Precision.HIGHEST)


def _relu(x):
  return jnp.maximum(x, 0.0)


def _mlp_pool_body(g_ref, c_ref, w0_ref, b0_ref, w1_ref, b1_ref, w2_ref,
                   b2_ref, out_ref, *, in_dim, out_pad):
  Mb, K, Dg = g_ref.shape[1], g_ref.shape[2], g_ref.shape[3]
  g = g_ref[0]
  cc = c_ref[0][:, :3]
  coords = g[:, :, :3] - cc[:, None, :]
  if in_dim > 3:
    feats = g[:, :, 3:in_dim]
    x = jnp.concatenate([coords, feats], axis=-1)
  else:
    x = coords
  x = x.reshape(Mb * K, in_dim)
  h = _relu(_dot_t(x, w0_ref[...]) + b0_ref[...])
  h = _relu(_dot_t(h, w1_ref[...]) + b1_ref[...])
  h = _relu(_dot_t(h, w2_ref[...]) + b2_ref[...])
  f = jnp.max(h.reshape(Mb, K, h.shape[-1]), axis=1)
  if out_pad:
    out = jnp.concatenate(
        [cc, f, jnp.zeros((Mb, out_pad), F32)], axis=-1)
  else:
    out = f
  out_ref[0] = out


def _mlp_pool(g, c, params, in_dim, out_pad, mb):
  B, M, K, Dg = g.shape
  Dc = c.shape[2]
  w0, b0, w1, b1, w2, b2 = params
  o = w2.shape[0]
  dout = (3 + o + out_pad) if out_pad else o
  body = functools.partial(_mlp_pool_body, in_dim=in_dim, out_pad=out_pad)
  wspec = lambda a: pl.BlockSpec(a.shape, lambda b, m: (0,) * a.ndim)
  return pl.pallas_call(
      body,
      grid=(B, M // mb),
      in_specs=[
          pl.BlockSpec((1, mb, K, Dg), lambda b, m: (b, m, 0, 0)),
          pl.BlockSpec((1, mb, Dc), lambda b, m: (b, m, 0)),
          wspec(w0), wspec(b0), wspec(w1), wspec(b1), wspec(w2), wspec(b2),
      ],
      out_specs=pl.BlockSpec((1, mb, dout), lambda b, m: (b, m, 0)),
      out_shape=jax.ShapeDtypeStruct((B, M, dout), F32),
      compiler_params=pltpu.CompilerParams(
          dimension_semantics=("parallel", "parallel")),
  )(g, c, w0, b0, w1, b1, w2, b2)


def _global_mlp_body(c_ref, f_ref, w0_ref, b0_ref, w1_ref, b1_ref, w2_ref,
                     b2_ref, out_ref):
  x = jnp.concatenate([c_ref[0][:, :3], f_ref[0]], axis=-1)
  h = _relu(_dot_t(x, w0_ref[...]) + b0_ref[...])
  h = _relu(_dot_t(h, w1_ref[...]) + b1_ref[...])
  h = _relu(_dot_t(h, w2_ref[...]) + b2_ref[...])
  out_ref[0, 0] = jnp.max(h, axis=0)


def _global_mlp(c, f, params):
  B, M, Dc = c.shape
  Fd = f.shape[2]
  w0, b0, w1, b1, w2, b2 = params
  o = w2.shape[0]
  wspec = lambda a: pl.BlockSpec(a.shape, lambda b: (0,) * a.ndim)
  return pl.pallas_call(
      _global_mlp_body,
      grid=(B,),
      in_specs=[
          pl.BlockSpec((1, M, Dc), lambda b: (b, 0, 0)),
          pl.BlockSpec((1, M, Fd), lambda b: (b, 0, 0)),
          wspec(w0), wspec(b0), wspec(w1), wspec(b1), wspec(w2), wspec(b2),
      ],
      out_specs=pl.BlockSpec((1, 1, o), lambda b: (b, 0, 0)),
      out_shape=jax.ShapeDtypeStruct((B, 1, o), F32),
      compiler_params=pltpu.CompilerParams(
          dimension_semantics=("parallel",)),
  )(c, f, w0, b0, w1, b1, w2, b2).reshape(B, o)


def kernel(x, sa1_W0, sa1_b0, sa1_W1, sa1_b1, sa1_W2, sa1_b2,
           sa2_W0, sa2_b0, sa2_W1, sa2_b1, sa2_W2, sa2_b2,
           sa3_W0, sa3_b0, sa3_W1, sa3_b1, sa3_W2, sa3_b2):
  B, N, _ = x.shape
  M1, K1, M2, K2 = 512, 32, 128, 64
  r1sq, r2sq = 0.2 * 0.2, 0.4 * 0.4

  p1 = (sa1_W0, sa1_b0.reshape(1, -1), sa1_W1, sa1_b1.reshape(1, -1),
        sa1_W2, sa1_b2.reshape(1, -1))
  p2 = (sa2_W0, sa2_b0.reshape(1, -1), sa2_W1, sa2_b1.reshape(1, -1),
        sa2_W2, sa2_b2.reshape(1, -1))
  p3 = (sa3_W0, sa3_b0.reshape(1, -1), sa3_W1, sa3_b1.reshape(1, -1),
        sa3_W2, sa3_b2.reshape(1, -1))

  xt = jnp.transpose(x, (0, 2, 1))
  x_pad = jnp.pad(x, ((0, 0), (0, 0), (0, 125))).reshape(B * N, 128)

  i1 = _fps(xt, M1)
  c1 = _sc_gather(x_pad, i1.reshape(-1))
  c1r = c1.reshape(B, M1, 128)
  idx1 = _ball_query(xt, c1r, K1, r1sq, 128)
  g1 = _sc_gather(x_pad, idx1.reshape(-1))
  g1 = g1.reshape(B, M1, K1, 128)
  cf1 = _mlp_pool(g1, c1r, p1, in_dim=3, out_pad=125, mb=128)
  cf1_flat = cf1.reshape(B * M1, 256)

  c1t = jnp.transpose(c1r[:, :, :3], (0, 2, 1))
  i2 = _fps(c1t, M2)
  c2 = _sc_gather(cf1_flat, i2.reshape(-1))
  c2r = c2.reshape(B, M2, 256)
  idx2 = _ball_query(c1t, c2r, K2, r2sq, 128)
  g2 = _sc_gather(cf1_flat, idx2.reshape(-1))
  g2 = g2.reshape(B, M2, K2, 256)
  f2 = _mlp_pool(g2, c2r, p2, in_dim=131, out_pad=0, mb=128)

  return _global_mlp(c2r, f2, p3)


# speedup vs baseline: 15.7521x; 1.3352x over previous
import functools

import jax
import jax.numpy as jnp
from jax import lax
from jax.experimental import pallas as pl
from jax.experimental.pallas import tpu as pltpu
from jax.experimental.pallas import tpu_sc as plsc

F32 = jnp.float32
_SC_CORES = 2
_SC_SUBCORES = 16
_SC_WORKERS = _SC_CORES * _SC_SUBCORES


def _fps_body(xt_ref, out_ref, lane_ref, col_ref):
  B, _, N = xt_ref.shape
  M = out_ref.shape[1]
  xs = xt_ref[:, 0, :]
  ys = xt_ref[:, 1, :]
  zs = xt_ref[:, 2, :]
  lane_ref[...] = lax.broadcasted_iota(jnp.int32, (B, N), 1)
  col_ref[...] = lax.broadcasted_iota(jnp.int32, (B, M), 1)
  lane = lane_ref[...]
  col = col_ref[...]

  def step(s, carry):
    far, dists, out = carry
    sel = (col == s).astype(jnp.int32)
    out = out + sel * (far - out)
    oh = (lane == far).astype(F32)
    cx = jnp.sum(xs * oh, axis=1, keepdims=True)
    cy = jnp.sum(ys * oh, axis=1, keepdims=True)
    cz = jnp.sum(zs * oh, axis=1, keepdims=True)
    d = (xs - cx) ** 2 + (ys - cy) ** 2 + (zs - cz) ** 2
    dists = jnp.minimum(dists, d)
    mx = jnp.max(dists, axis=1, keepdims=True)
    far = jnp.min(lane + (dists != mx).astype(jnp.int32) * N,
                  axis=1, keepdims=True)
    return far, dists, out

  far0 = col[:, :1] * 0
  d0 = jnp.maximum(xs * 0.0, 1e10)
  out0 = col * 0
  _, _, out = lax.fori_loop(0, M, step, (far0, d0, out0))
  col_ref[...] = lax.broadcasted_iota(jnp.int32, (B, M), 0)
  out_ref[...] = out + col_ref[...] * N


def _fps(xt, m):
  B, _, N = xt.shape
  return pl.pallas_call(
      _fps_body,
      out_shape=jax.ShapeDtypeStruct((B, m), jnp.int32),
      scratch_shapes=[
          pltpu.VMEM((B, N), jnp.int32),
          pltpu.VMEM((B, m), jnp.int32),
      ],
  )(xt)


def _bq_body(k, r2, xt_ref, c_ref, out_ref, lane_ref):
  N = xt_ref.shape[2]
  Mb = c_ref.shape[1]
  b = pl.program_id(0)
  xs = xt_ref[0, 0, :][None, :]
  ys = xt_ref[0, 1, :][None, :]
  zs = xt_ref[0, 2, :][None, :]
  cx = c_ref[0, :, 0:1]
  cy = c_ref[0, :, 1:2]
  cz = c_ref[0, :, 2:3]
  d = (cx - xs) ** 2 + (cy - ys) ** 2 + (cz - zs) ** 2
  lane_ref[...] = lax.broadcasted_iota(jnp.int32, (Mb, N), 1)
  lane = lane_ref[...].astype(F32)
  midx = lane + (d > r2).astype(F32) * N
  big = 3.0 * N
  cur = jnp.min(midx, axis=1, keepdims=True)
  cols = [cur]
  for _ in range(k - 1):
    cur = jnp.min(jnp.where(midx > cur, midx, big), axis=1, keepdims=True)
    cols.append(cur)
  idx = jnp.concatenate(cols, axis=1)
  pad = (idx >= N).astype(F32)
  idx = idx + pad * (idx[:, 0:1] - idx)
  out_ref[0] = idx.astype(jnp.int32) + b * N


def _ball_query(xt, c, k, r2, mb):
  B, _, N = xt.shape
  M = c.shape[1]
  Dc = c.shape[2]
  body = functools.partial(_bq_body, k, r2)
  return pl.pallas_call(
      body,
      grid=(B, M // mb),
      in_specs=[
          pl.BlockSpec((1, 3, N), lambda b, m: (b, 0, 0)),
          pl.BlockSpec((1, mb, Dc), lambda b, m: (b, m, 0)),
      ],
      out_specs=pl.BlockSpec((1, mb, k), lambda b, m: (b, m, 0)),
      out_shape=jax.ShapeDtypeStruct((B, M, k), jnp.int32),
      scratch_shapes=[pltpu.VMEM((mb, N), jnp.int32)],
      compiler_params=pltpu.CompilerParams(
          dimension_semantics=("parallel", "parallel")),
  )(xt, c)


def _sc_gather(table, idx):
  R = idx.shape[0]
  D = table.shape[1]
  bpw = R // _SC_WORKERS
  ch = min(bpw, 128)
  nch = bpw // ch
  mesh = plsc.VectorSubcoreMesh(core_axis_name="c", subcore_axis_name="s")

  @functools.partial(
      pl.kernel,
      mesh=mesh,
      out_type=jax.ShapeDtypeStruct((R, D), F32),
      scratch_types=[
          pltpu.VMEM((ch,), jnp.int32),
          pltpu.VMEM((ch, D), F32),
          pltpu.SemaphoreType.DMA,
      ],
  )
  def k(table_hbm, idx_hbm, out_hbm, idx_v, rows_v, sem):
    wid = lax.axis_index("s") * _SC_CORES + lax.axis_index("c")
    base = wid * bpw

    def body(i, carry):
      off = pl.multiple_of(base + i * ch, 8)
      pltpu.sync_copy(idx_hbm.at[pl.ds(off, ch)], idx_v)
      pltpu.async_copy(table_hbm.at[idx_v], rows_v, sem).wait()
      pltpu.sync_copy(rows_v, out_hbm.at[pl.ds(off, ch)])
      return carry

    lax.fori_loop(0, nch, body, 0)

  return k(table, idx)


def _dot_t(x, w):
  return lax.dot_general(x, w, (((1,), (1,)), ((), ())))


def _relu(x):
  return jnp.maximum(x, 0.0)


def _mlp_pool_body(g_ref, c_ref, w0_ref, b0_ref, w1_ref, b1_ref, w2_ref,
                   b2_ref, out_ref, *, in_dim, out_pad):
  Mb, K, Dg = g_ref.shape[1], g_ref.shape[2], g_ref.shape[3]
  g = g_ref[0]
  cc = c_ref[0][:, :3]
  coords = g[:, :, :3] - cc[:, None, :]
  if in_dim > 3:
    feats = g[:, :, 3:in_dim]
    x = jnp.concatenate([coords, feats], axis=-1)
  else:
    x = coords
  x = x.reshape(Mb * K, in_dim)
  h = _relu(_dot_t(x, w0_ref[...]) + b0_ref[...])
  h = _relu(_dot_t(h, w1_ref[...]) + b1_ref[...])
  h = _relu(_dot_t(h, w2_ref[...]) + b2_ref[...])
  f = jnp.max(h.reshape(Mb, K, h.shape[-1]), axis=1)
  if out_pad:
    out = jnp.concatenate(
        [cc, f, jnp.zeros((Mb, out_pad), F32)], axis=-1)
  else:
    out = f
  out_ref[0] = out


def _mlp_pool(g, c, params, in_dim, out_pad, mb):
  B, M, K, Dg = g.shape
  Dc = c.shape[2]
  w0, b0, w1, b1, w2, b2 = params
  o = w2.shape[0]
  dout = (3 + o + out_pad) if out_pad else o
  body = functools.partial(_mlp_pool_body, in_dim=in_dim, out_pad=out_pad)
  wspec = lambda a: pl.BlockSpec(a.shape, lambda b, m: (0,) * a.ndim)
  return pl.pallas_call(
      body,
      grid=(B, M // mb),
      in_specs=[
          pl.BlockSpec((1, mb, K, Dg), lambda b, m: (b, m, 0, 0)),
          pl.BlockSpec((1, mb, Dc), lambda b, m: (b, m, 0)),
          wspec(w0), wspec(b0), wspec(w1), wspec(b1), wspec(w2), wspec(b2),
      ],
      out_specs=pl.BlockSpec((1, mb, dout), lambda b, m: (b, m, 0)),
      out_shape=jax.ShapeDtypeStruct((B, M, dout), F32),
      compiler_params=pltpu.CompilerParams(
          dimension_semantics=("parallel", "parallel")),
  )(g, c, w0, b0, w1, b1, w2, b2)


def _global_mlp_body(c_ref, f_ref, w0_ref, b0_ref, w1_ref, b1_ref, w2_ref,
                     b2_ref, out_ref):
  x = jnp.concatenate([c_ref[0][:, :3], f_ref[0]], axis=-1)
  h = _relu(_dot_t(x, w0_ref[...]) + b0_ref[...])
  h = _relu(_dot_t(h, w1_ref[...]) + b1_ref[...])
  h = _relu(_dot_t(h, w2_ref[...]) + b2_ref[...])
  out_ref[0, 0] = jnp.max(h, axis=0)


def _global_mlp(c, f, params):
  B, M, Dc = c.shape
  Fd = f.shape[2]
  w0, b0, w1, b1, w2, b2 = params
  o = w2.shape[0]
  wspec = lambda a: pl.BlockSpec(a.shape, lambda b: (0,) * a.ndim)
  return pl.pallas_call(
      _global_mlp_body,
      grid=(B,),
      in_specs=[
          pl.BlockSpec((1, M, Dc), lambda b: (b, 0, 0)),
          pl.BlockSpec((1, M, Fd), lambda b: (b, 0, 0)),
          wspec(w0), wspec(b0), wspec(w1), wspec(b1), wspec(w2), wspec(b2),
      ],
      out_specs=pl.BlockSpec((1, 1, o), lambda b: (b, 0, 0)),
      out_shape=jax.ShapeDtypeStruct((B, 1, o), F32),
      compiler_params=pltpu.CompilerParams(
          dimension_semantics=("parallel",)),
  )(c, f, w0, b0, w1, b1, w2, b2).reshape(B, o)


def kernel(x, sa1_W0, sa1_b0, sa1_W1, sa1_b1, sa1_W2, sa1_b2,
           sa2_W0, sa2_b0, sa2_W1, sa2_b1, sa2_W2, sa2_b2,
           sa3_W0, sa3_b0, sa3_W1, sa3_b1, sa3_W2, sa3_b2):
  B, N, _ = x.shape
  M1, K1, M2, K2 = 512, 32, 128, 64
  r1sq, r2sq = 0.2 * 0.2, 0.4 * 0.4

  p1 = (sa1_W0, sa1_b0.reshape(1, -1), sa1_W1, sa1_b1.reshape(1, -1),
        sa1_W2, sa1_b2.reshape(1, -1))
  p2 = (sa2_W0, sa2_b0.reshape(1, -1), sa2_W1, sa2_b1.reshape(1, -1),
        sa2_W2, sa2_b2.reshape(1, -1))
  p3 = (sa3_W0, sa3_b0.reshape(1, -1), sa3_W1, sa3_b1.reshape(1, -1),
        sa3_W2, sa3_b2.reshape(1, -1))

  xt = jnp.transpose(x, (0, 2, 1))
  x_pad = jnp.pad(x, ((0, 0), (0, 0), (0, 125))).reshape(B * N, 128)

  i1 = _fps(xt, M1)
  c1 = _sc_gather(x_pad, i1.reshape(-1))
  c1r = c1.reshape(B, M1, 128)
  idx1 = _ball_query(xt, c1r, K1, r1sq, 128)
  g1 = _sc_gather(x_pad, idx1.reshape(-1))
  g1 = g1.reshape(B, M1, K1, 128)
  cf1 = _mlp_pool(g1, c1r, p1, in_dim=3, out_pad=125, mb=128)
  cf1_flat = cf1.reshape(B * M1, 256)

  c1t = jnp.transpose(c1r[:, :, :3], (0, 2, 1))
  i2 = _fps(c1t, M2)
  c2 = _sc_gather(cf1_flat, i2.reshape(-1))
  c2r = c2.reshape(B, M2, 256)
  idx2 = _ball_query(c1t, c2r, K2, r2sq, 128)
  g2 = _sc_gather(cf1_flat, idx2.reshape(-1))
  g2 = g2.reshape(B, M2, K2, 256)
  f2 = _mlp_pool(g2, c2r, p2, in_dim=131, out_pad=0, mb=128)

  return _global_mlp(c2r, f2, p3)
